# baseline (speedup 1.0000x reference)
import functools

import jax
import jax.numpy as jnp
import numpy as np
from jax import lax
from jax.experimental import pallas as pl
from jax.experimental.pallas import tpu as pltpu
from jax.experimental.pallas import tpu_sc as plsc

_D = 64
_ROW = 2 * _D
_EG = 0.57721566490153286
_EPS = 1e-23
_NC, _NS = 2, 16
_NW = _NC * _NS
_GCHUNK = 128


def _sc_gather(gci_flat, table):
    rows_total = gci_flat.shape[0]
    r_per_w = rows_total // _NW
    n_chunks = r_per_w // _GCHUNK
    mesh = plsc.VectorSubcoreMesh(core_axis_name="c", subcore_axis_name="s")

    @functools.partial(
        pl.kernel,
        out_type=jax.ShapeDtypeStruct((rows_total, _ROW), jnp.float32),
        mesh=mesh,
        scratch_types=[
            pltpu.VMEM((r_per_w,), jnp.int32),
            pltpu.VMEM((_GCHUNK, _ROW), jnp.float32),
            pltpu.VMEM((_GCHUNK, _ROW), jnp.float32),
            pltpu.SemaphoreType.DMA,
            pltpu.SemaphoreType.DMA,
        ],
    )
    def gather_kernel(gci_hbm, table_hbm, out_hbm, idx_v, rows_a, rows_b, sem_a, sem_b):
        wid = lax.axis_index("s") * _NC + lax.axis_index("c")
        base = wid * r_per_w
        pltpu.sync_copy(gci_hbm.at[pl.ds(base, r_per_w)], idx_v)
        bufs = ((rows_a, sem_a), (rows_b, sem_b))
        pltpu.async_copy(
            table_hbm.at[idx_v.at[pl.ds(0, _GCHUNK)]], rows_a, sem_a)
        for j in range(n_chunks):
            buf, sem = bufs[j % 2]
            nbuf, nsem = bufs[(j + 1) % 2]
            if j + 1 < n_chunks:
                pltpu.async_copy(
                    table_hbm.at[idx_v.at[pl.ds((j + 1) * _GCHUNK, _GCHUNK)]],
                    nbuf, nsem)
            pltpu.make_async_copy(
                table_hbm.at[idx_v.at[pl.ds(j * _GCHUNK, _GCHUNK)]], buf, sem
            ).wait()
            pltpu.sync_copy(buf, out_hbm.at[pl.ds(base + j * _GCHUNK, _GCHUNK)])

    return gather_kernel(gci_flat, table)


def _tc_compute(pairs):
    batch = pairs.shape[0]
    blk = 2048
    grid = batch // blk
    inv_k = float(np.exp(-2.0 * _EG))

    def body(x_ref, o_ref):
        x = x_ref[...]
        z_sub = x[:, 0 * _D:1 * _D]
        d_sub = x[:, 1 * _D:2 * _D]
        z_sup = x[:, 2 * _D:3 * _D]
        d_sup = x[:, 3 * _D:4 * _D]
        ea = jnp.exp(z_sub)
        eb = jnp.exp(z_sup)
        pda = 1.0 + jnp.exp(d_sub)
        pdb = 1.0 + jnp.exp(d_sup)
        big_a = ea * pda
        big_b = eb * pdb
        s = ea + eb
        t = big_a + big_b
        pm = big_a * big_b
        num = jnp.log1p(pm / (t * s) * inv_k) + _EPS
        den = jnp.log1p(pda * inv_k) + _EPS
        lsum = jnp.sum(jnp.log(num / den), axis=-1)
        o_ref[...] = jnp.clip(jnp.exp(lsum), 0.0, 1.0)

    return pl.pallas_call(
        body,
        grid=(grid,),
        in_specs=[pl.BlockSpec((blk, 4 * _D), lambda i: (i, 0))],
        out_specs=pl.BlockSpec((blk,), lambda i: (i,)),
        out_shape=jax.ShapeDtypeStruct((batch,), jnp.float32),
    )(pairs)


def _tc_trivial(pairs):
    batch = pairs.shape[0]
    blk = 2048
    grid = batch // blk

    def body(x_ref, o_ref):
        o_ref[...] = jnp.sum(x_ref[...], axis=-1)

    return pl.pallas_call(
        body,
        grid=(grid,),
        in_specs=[pl.BlockSpec((blk, 4 * _D), lambda i: (i, 0))],
        out_specs=pl.BlockSpec((blk,), lambda i: (i,)),
        out_shape=jax.ShapeDtypeStruct((batch,), jnp.float32),
    )(pairs)


def kernel(gci, table):
    batch = gci.shape[0]
    nsplit = 2
    gci_flat = gci.reshape(nsplit, -1)
    outs = []
    for s in range(nsplit):
        gathered = _sc_gather(gci_flat[s], table)
        outs.append(_tc_compute(gathered.reshape(batch // nsplit, 2 * _ROW)))
    return jnp.concatenate(outs)


# speedup vs baseline: 1.0666x; 1.0666x over previous
import functools

import jax
import jax.numpy as jnp
import numpy as np
from jax import lax
from jax.experimental import pallas as pl
from jax.experimental.pallas import tpu as pltpu
from jax.experimental.pallas import tpu_sc as plsc

_D = 64
_ROW = 2 * _D
_EG = 0.57721566490153286
_EPS = 1e-23
_NC, _NS = 2, 16
_NW = _NC * _NS
_GCHUNK = 128


def _sc_gather(gci_flat, table):
    rows_total = gci_flat.shape[0]
    r_per_w = rows_total // _NW
    n_chunks = r_per_w // _GCHUNK
    mesh = plsc.VectorSubcoreMesh(core_axis_name="c", subcore_axis_name="s")

    @functools.partial(
        pl.kernel,
        out_type=jax.ShapeDtypeStruct((rows_total, _ROW), jnp.float32),
        mesh=mesh,
        scratch_types=[
            pltpu.VMEM((r_per_w,), jnp.int32),
            pltpu.VMEM((_GCHUNK, _ROW), jnp.float32),
            pltpu.VMEM((_GCHUNK, _ROW), jnp.float32),
            pltpu.SemaphoreType.DMA,
            pltpu.SemaphoreType.DMA,
        ],
    )
    def gather_kernel(gci_hbm, table_hbm, out_hbm, idx_v, rows_a, rows_b, sem_a, sem_b):
        wid = lax.axis_index("s") * _NC + lax.axis_index("c")
        base = wid * r_per_w
        pltpu.sync_copy(gci_hbm.at[pl.ds(base, r_per_w)], idx_v)
        bufs = ((rows_a, sem_a), (rows_b, sem_b))
        pltpu.async_copy(
            table_hbm.at[idx_v.at[pl.ds(0, _GCHUNK)]], rows_a, sem_a)
        for j in range(n_chunks):
            buf, sem = bufs[j % 2]
            nbuf, nsem = bufs[(j + 1) % 2]
            if j + 1 < n_chunks:
                pltpu.async_copy(
                    table_hbm.at[idx_v.at[pl.ds((j + 1) * _GCHUNK, _GCHUNK)]],
                    nbuf, nsem)
            pltpu.make_async_copy(
                table_hbm.at[idx_v.at[pl.ds(j * _GCHUNK, _GCHUNK)]], buf, sem
            ).wait()
            pltpu.sync_copy(buf, out_hbm.at[pl.ds(base + j * _GCHUNK, _GCHUNK)])

    return gather_kernel(gci_flat, table)


def _tc_compute(pairs):
    batch = pairs.shape[0]
    blk = 2048
    grid = batch // blk
    inv_k = float(np.exp(-2.0 * _EG))

    def body(x_ref, o_ref):
        x = x_ref[...]
        z_sub = x[:, 0 * _D:1 * _D]
        d_sub = x[:, 1 * _D:2 * _D]
        z_sup = x[:, 2 * _D:3 * _D]
        d_sup = x[:, 3 * _D:4 * _D]
        ea = jnp.exp(z_sub)
        eb = jnp.exp(z_sup)
        pda = 1.0 + jnp.exp(d_sub)
        pdb = 1.0 + jnp.exp(d_sup)
        big_a = ea * pda
        big_b = eb * pdb
        s = ea + eb
        t = big_a + big_b
        pm = big_a * big_b
        num = jnp.log1p(pm / (t * s) * inv_k) + _EPS
        den = jnp.log1p(pda * inv_k) + _EPS
        lsum = jnp.sum(jnp.log(num / den), axis=-1)
        o_ref[...] = jnp.clip(jnp.exp(lsum), 0.0, 1.0)

    return pl.pallas_call(
        body,
        grid=(grid,),
        in_specs=[pl.BlockSpec((blk, 4 * _D), lambda i: (i, 0))],
        out_specs=pl.BlockSpec((blk,), lambda i: (i,)),
        out_shape=jax.ShapeDtypeStruct((batch,), jnp.float32),
    )(pairs)


def _tc_trivial(pairs):
    batch = pairs.shape[0]
    blk = 2048
    grid = batch // blk

    def body(x_ref, o_ref):
        o_ref[...] = jnp.sum(x_ref[...], axis=-1)

    return pl.pallas_call(
        body,
        grid=(grid,),
        in_specs=[pl.BlockSpec((blk, 4 * _D), lambda i: (i, 0))],
        out_specs=pl.BlockSpec((blk,), lambda i: (i,)),
        out_shape=jax.ShapeDtypeStruct((batch,), jnp.float32),
    )(pairs)


def kernel(gci, table):
    batch = gci.shape[0]
    gathered = _sc_gather(gci.reshape(-1), table)
    return _tc_compute(gathered.reshape(batch, 2 * _ROW))


# speedup vs baseline: 1.3409x; 1.2571x over previous
import functools

import jax
import jax.numpy as jnp
import numpy as np
from jax import lax
from jax.experimental import pallas as pl
from jax.experimental.pallas import tpu as pltpu
from jax.experimental.pallas import tpu_sc as plsc

_D = 64
_ROW = 2 * _D
_EG = 0.57721566490153286
_EPS = 1e-23
_NC, _NS = 2, 16
_NW = _NC * _NS
_GCHUNK = 128


def _tc_index_prep(gci):
    batch = gci.shape[0]
    blk = 2048

    def body(g_ref, sub_ref, sup_ref):
        g = g_ref[...]
        sub_ref[...] = g[:, 0].reshape(blk // _GCHUNK, _GCHUNK)
        sup_ref[...] = g[:, 1].reshape(blk // _GCHUNK, _GCHUNK)

    grid = batch // blk
    return pl.pallas_call(
        body,
        grid=(grid,),
        in_specs=[pl.BlockSpec((blk, 2), lambda i: (i, 0))],
        out_specs=[
            pl.BlockSpec((blk // _GCHUNK, _GCHUNK), lambda i: (i, 0)),
            pl.BlockSpec((blk // _GCHUNK, _GCHUNK), lambda i: (i, 0)),
        ],
        out_shape=[
            jax.ShapeDtypeStruct((batch // _GCHUNK, _GCHUNK), jnp.int32),
            jax.ShapeDtypeStruct((batch // _GCHUNK, _GCHUNK), jnp.int32),
        ],
    )(gci)


def _sc_gather(sub_idx, sup_idx, table):
    batch = sub_idx.shape[0] * _GCHUNK
    e_per_w = batch // _NW
    n_chunks = e_per_w // _GCHUNK
    mesh = plsc.VectorSubcoreMesh(core_axis_name="c", subcore_axis_name="s")

    @functools.partial(
        pl.kernel,
        out_type=jax.ShapeDtypeStruct((2 * batch, _ROW), jnp.float32),
        mesh=mesh,
        scratch_types=[
            pltpu.VMEM((n_chunks, _GCHUNK), jnp.int32),
            pltpu.VMEM((n_chunks, _GCHUNK), jnp.int32),
            pltpu.VMEM((_GCHUNK, _ROW), jnp.float32),
            pltpu.VMEM((_GCHUNK, _ROW), jnp.float32),
            pltpu.SemaphoreType.DMA,
            pltpu.SemaphoreType.DMA,
        ],
    )
    def gather_kernel(sub_hbm, sup_hbm, table_hbm, out_hbm, isub_v, isup_v,
                      rows_a, rows_b, sem_a, sem_b):
        wid = lax.axis_index("s") * _NC + lax.axis_index("c")
        base = wid * e_per_w
        row0 = wid * n_chunks
        pltpu.sync_copy(sub_hbm.at[pl.ds(row0, n_chunks), :], isub_v)
        pltpu.sync_copy(sup_hbm.at[pl.ds(row0, n_chunks), :], isup_v)
        jobs = []
        for j in range(n_chunks):
            jobs.append((isub_v.at[j], base + j * _GCHUNK))
        for j in range(n_chunks):
            jobs.append((isup_v.at[j], batch + base + j * _GCHUNK))
        bufs = ((rows_a, sem_a), (rows_b, sem_b))
        pltpu.async_copy(table_hbm.at[jobs[0][0]], rows_a, sem_a)
        for j, (idx_ref, dst_off) in enumerate(jobs):
            buf, sem = bufs[j % 2]
            nbuf, nsem = bufs[(j + 1) % 2]
            if j + 1 < len(jobs):
                pltpu.async_copy(table_hbm.at[jobs[j + 1][0]], nbuf, nsem)
            pltpu.make_async_copy(table_hbm.at[idx_ref], buf, sem).wait()
            pltpu.sync_copy(buf, out_hbm.at[pl.ds(dst_off, _GCHUNK)])

    return gather_kernel(sub_idx, sup_idx, table)


def _tc_compute(staging, batch):
    blk = 2048
    grid = batch // blk
    inv_k = float(np.exp(-2.0 * _EG))

    def body(sub_ref, sup_ref, o_ref):
        sub = sub_ref[...]
        sup = sup_ref[...]
        ea = jnp.exp(sub[:, :_D])
        eb = jnp.exp(sup[:, :_D])
        pda = 1.0 + jnp.exp(sub[:, _D:])
        pdb = 1.0 + jnp.exp(sup[:, _D:])
        big_a = ea * pda
        big_b = eb * pdb
        s = ea + eb
        t = big_a + big_b
        pm = big_a * big_b
        num = jnp.log1p(pm / (t * s) * inv_k) + _EPS
        den = jnp.log1p(pda * inv_k) + _EPS
        lsum = jnp.sum(jnp.log(num / den), axis=-1)
        o_ref[...] = jnp.clip(jnp.exp(lsum), 0.0, 1.0)

    return pl.pallas_call(
        body,
        grid=(grid,),
        in_specs=[
            pl.BlockSpec((blk, _ROW), lambda i: (i, 0)),
            pl.BlockSpec((blk, _ROW), lambda i: (i + grid, 0)),
        ],
        out_specs=pl.BlockSpec((blk,), lambda i: (i,)),
        out_shape=jax.ShapeDtypeStruct((batch,), jnp.float32),
    )(staging, staging)


def kernel(gci, table):
    batch = gci.shape[0]
    sub_idx, sup_idx = _tc_index_prep(gci)
    staging = _sc_gather(sub_idx, sup_idx, table)
    return _tc_compute(staging, batch)


# speedup vs baseline: 1.3792x; 1.0286x over previous
import functools

import jax
import jax.numpy as jnp
import numpy as np
from jax import lax
from jax.experimental import pallas as pl
from jax.experimental.pallas import tpu as pltpu
from jax.experimental.pallas import tpu_sc as plsc

_D = 64
_ROW = 2 * _D
_EG = 0.57721566490153286
_EPS = 1e-23
_NC, _NS = 2, 16
_NW = _NC * _NS
_GCHUNK = 128
_L = 16


def _sc_gather(gci2d, table):
    batch = gci2d.shape[0] * 64
    e_per_w = batch // _NW
    n_chunks = e_per_w // _GCHUNK
    gci_rows_per_w = 2 * e_per_w // 128
    mesh = plsc.VectorSubcoreMesh(core_axis_name="c", subcore_axis_name="s")

    @functools.partial(
        pl.kernel,
        out_type=jax.ShapeDtypeStruct((2 * batch, _ROW), jnp.float32),
        mesh=mesh,
        scratch_types=[
            pltpu.VMEM((gci_rows_per_w, 128), jnp.int32),
            pltpu.VMEM((e_per_w,), jnp.int32),
            pltpu.VMEM((e_per_w,), jnp.int32),
            pltpu.VMEM((_GCHUNK, _ROW), jnp.float32),
            pltpu.VMEM((_GCHUNK, _ROW), jnp.float32),
            pltpu.SemaphoreType.DMA,
            pltpu.SemaphoreType.DMA,
        ],
    )
    def gather_kernel(gci_hbm, table_hbm, out_hbm, pairs_v, isub_v, isup_v,
                      rows_a, rows_b, sem_a, sem_b):
        wid = lax.axis_index("s") * _NC + lax.axis_index("c")
        base = wid * e_per_w
        pltpu.sync_copy(
            gci_hbm.at[pl.ds(wid * gci_rows_per_w, gci_rows_per_w), :],
            pairs_v)
        iota = lax.iota(jnp.int32, _L)
        gidx = (2 * iota) % _L
        half = iota < 8
        for off in range(0, e_per_w, _L):
            flat = 2 * off
            a = pairs_v[flat // 128, pl.ds(flat % 128, _L)]
            b = pairs_v[flat // 128, pl.ds(flat % 128 + _L, _L)]
            ga = a.at[gidx].get(mode="promise_in_bounds")
            gb = b.at[gidx].get(mode="promise_in_bounds")
            isub_v[pl.ds(off, _L)] = jnp.where(half, ga, gb)
            ga1 = a.at[gidx + 1].get(mode="promise_in_bounds")
            gb1 = b.at[gidx + 1].get(mode="promise_in_bounds")
            isup_v[pl.ds(off, _L)] = jnp.where(half, ga1, gb1)
        jobs = []
        for j in range(n_chunks):
            jobs.append((isub_v.at[pl.ds(j * _GCHUNK, _GCHUNK)],
                         base + j * _GCHUNK))
        for j in range(n_chunks):
            jobs.append((isup_v.at[pl.ds(j * _GCHUNK, _GCHUNK)],
                         batch + base + j * _GCHUNK))
        bufs = ((rows_a, sem_a), (rows_b, sem_b))
        pltpu.async_copy(table_hbm.at[jobs[0][0]], rows_a, sem_a)
        for j, (idx_ref, dst_off) in enumerate(jobs):
            buf, sem = bufs[j % 2]
            nbuf, nsem = bufs[(j + 1) % 2]
            if j + 1 < len(jobs):
                pltpu.async_copy(table_hbm.at[jobs[j + 1][0]], nbuf, nsem)
            pltpu.make_async_copy(table_hbm.at[idx_ref], buf, sem).wait()
            pltpu.sync_copy(buf, out_hbm.at[pl.ds(dst_off, _GCHUNK)])

    return gather_kernel(gci2d, table)


def _tc_compute(staging, batch):
    blk = 2048
    grid = batch // blk
    inv_k = float(np.exp(-2.0 * _EG))

    def body(sub_ref, sup_ref, o_ref):
        sub = sub_ref[...]
        sup = sup_ref[...]
        ea = jnp.exp(sub[:, :_D])
        eb = jnp.exp(sup[:, :_D])
        pda = 1.0 + jnp.exp(sub[:, _D:])
        pdb = 1.0 + jnp.exp(sup[:, _D:])
        big_a = ea * pda
        big_b = eb * pdb
        s = ea + eb
        t = big_a + big_b
        pm = big_a * big_b
        num = jnp.log1p(pm / (t * s) * inv_k) + _EPS
        den = jnp.log1p(pda * inv_k) + _EPS
        lsum = jnp.sum(jnp.log(num / den), axis=-1)
        o_ref[...] = jnp.clip(jnp.exp(lsum), 0.0, 1.0)

    return pl.pallas_call(
        body,
        grid=(grid,),
        in_specs=[
            pl.BlockSpec((blk, _ROW), lambda i: (i, 0)),
            pl.BlockSpec((blk, _ROW), lambda i: (i + grid, 0)),
        ],
        out_specs=pl.BlockSpec((blk,), lambda i: (i,)),
        out_shape=jax.ShapeDtypeStruct((batch,), jnp.float32),
    )(staging, staging)


def kernel(gci, table):
    batch = gci.shape[0]
    staging = _sc_gather(gci.reshape(batch // 64, 128), table)
    return _tc_compute(staging, batch)


# speedup vs baseline: 1.6320x; 1.1833x over previous
import functools

import jax
import jax.numpy as jnp
import numpy as np
from jax import lax
from jax.experimental import pallas as pl
from jax.experimental.pallas import tpu as pltpu
from jax.experimental.pallas import tpu_sc as plsc

_D = 64
_ROW = 2 * _D
_EG = 0.57721566490153286
_EPS = 1e-23
_NC, _NS = 2, 16
_NW = _NC * _NS
_GCHUNK = 128
_L = 16


def _sc_gather(idx2d, table):
    batch = idx2d.shape[0] * 64
    e_per_w = batch // _NW
    n_chunks = e_per_w // _GCHUNK
    idx_rows_per_w = 2 * e_per_w // 128
    mesh = plsc.VectorSubcoreMesh(core_axis_name="c", subcore_axis_name="s")

    @functools.partial(
        pl.kernel,
        out_type=jax.ShapeDtypeStruct((2 * batch, _ROW), jnp.float32),
        mesh=mesh,
        scratch_types=[
            pltpu.VMEM((idx_rows_per_w, 128), jnp.int32),
            pltpu.VMEM((_GCHUNK, _ROW), jnp.float32),
            pltpu.VMEM((_GCHUNK, _ROW), jnp.float32),
            pltpu.SemaphoreType.DMA,
            pltpu.SemaphoreType.DMA,
        ],
    )
    def gather_kernel(idx_hbm, table_hbm, out_hbm, pairs_v,
                      rows_a, rows_b, sem_a, sem_b):
        wid = lax.axis_index("s") * _NC + lax.axis_index("c")
        base = wid * e_per_w
        pltpu.sync_copy(
            idx_hbm.at[pl.ds(wid * idx_rows_per_w, idx_rows_per_w), :],
            pairs_v)
        jobs = []
        for j in range(n_chunks):
            jobs.append((pairs_v.at[2 * j], base + j * _GCHUNK))
        for j in range(n_chunks):
            jobs.append((pairs_v.at[2 * j + 1], batch + base + j * _GCHUNK))
        bufs = ((rows_a, sem_a), (rows_b, sem_b))
        pltpu.async_copy(table_hbm.at[jobs[0][0]], rows_a, sem_a)
        for j, (idx_ref, dst_off) in enumerate(jobs):
            buf, sem = bufs[j % 2]
            nbuf, nsem = bufs[(j + 1) % 2]
            if j + 1 < len(jobs):
                pltpu.async_copy(table_hbm.at[jobs[j + 1][0]], nbuf, nsem)
            pltpu.make_async_copy(table_hbm.at[idx_ref], buf, sem).wait()
            pltpu.sync_copy(buf, out_hbm.at[pl.ds(dst_off, _GCHUNK)])

    return gather_kernel(idx2d, table)


def _tc_compute(staging, batch):
    blk = 2048
    grid = batch // blk
    inv_k = float(np.exp(-2.0 * _EG))

    def body(sub_ref, sup_ref, o_ref):
        sub = sub_ref[...]
        sup = sup_ref[...]
        ea = jnp.exp(sub[:, :_D])
        eb = jnp.exp(sup[:, :_D])
        pda = 1.0 + jnp.exp(sub[:, _D:])
        pdb = 1.0 + jnp.exp(sup[:, _D:])
        big_a = ea * pda
        big_b = eb * pdb
        s = ea + eb
        t = big_a + big_b
        pm = big_a * big_b
        num = jnp.log1p(pm / (t * s) * inv_k) + _EPS
        den = jnp.log1p(pda * inv_k) + _EPS
        lsum = jnp.sum(jnp.log(num / den), axis=-1)
        o_ref[...] = jnp.clip(jnp.exp(lsum), 0.0, 1.0)

    return pl.pallas_call(
        body,
        grid=(grid,),
        in_specs=[
            pl.BlockSpec((blk, _ROW), lambda i: (i, 0)),
            pl.BlockSpec((blk, _ROW), lambda i: (i + grid, 0)),
        ],
        out_specs=pl.BlockSpec((blk,), lambda i: (i,)),
        out_shape=jax.ShapeDtypeStruct((batch,), jnp.float32),
    )(staging, staging)


def kernel(gci, table):
    batch = gci.shape[0]
    idx2d = (gci.T.reshape(2, batch // 128, 128)
             .transpose(1, 0, 2).reshape(batch // 64, 128))
    staging = _sc_gather(idx2d, table)
    return _tc_compute(staging, batch)
